# Initial kernel scaffold; baseline (speedup 1.0000x reference)
import jax
import jax.numpy as jnp
from jax.experimental import pallas as pl


def kernel(xyz, feature, raw_relative_feature, neighbors_idx, W_rel, b_rel, g_rel, be_rel, W_attn, W_out, b_out, g_out, be_out, W_sc, b_sc, g_sc, be_sc):
    raise NotImplementedError("write your pallas kernel here")


# speedup vs baseline: 6.1370x; 6.1370x over previous
import functools

import jax
import jax.numpy as jnp
from jax import lax
from jax.experimental import pallas as pl
from jax.experimental.pallas import tpu as pltpu
from jax.experimental.pallas import tpu_sc as plsc

_EPS = 1e-5
_SLOPE = 0.2
_F32 = jnp.float32


def _leaky(x):
    return jnp.maximum(x, _SLOPE * x)


@functools.lru_cache(maxsize=None)
def _make_sc_gather(n_idx: int, d: int):
    NC, NS = 2, 16
    NW = NC * NS
    per_w = n_idx // NW
    assert per_w * NW == n_idx and per_w % 8 == 0
    CH = 128
    nfull, tail = divmod(per_w, CH)
    mesh = plsc.VectorSubcoreMesh(core_axis_name="c", subcore_axis_name="s")
    scratch = [
        pltpu.VMEM((CH,), jnp.int32),
        pltpu.VMEM((CH, d), _F32),
        pltpu.SemaphoreType.DMA,
    ]
    if tail:
        scratch += [pltpu.VMEM((tail,), jnp.int32), pltpu.VMEM((tail, d), _F32)]

    def body(table_hbm, idx_hbm, out_hbm, idx_v, rows_v, sem, *ts):
        wid = lax.axis_index("s") * NC + lax.axis_index("c")
        base = wid * per_w

        def chunk(c, carry):
            off = base + c * CH
            pltpu.sync_copy(idx_hbm.at[pl.ds(off, CH)], idx_v)
            pltpu.async_copy(table_hbm.at[idx_v], rows_v, sem).wait()
            pltpu.sync_copy(rows_v, out_hbm.at[pl.ds(off, CH)])
            return carry

        lax.fori_loop(0, nfull, chunk, 0)
        if tail:
            idx_t, rows_t = ts
            off = base + nfull * CH
            pltpu.sync_copy(idx_hbm.at[pl.ds(off, tail)], idx_t)
            pltpu.async_copy(table_hbm.at[idx_t], rows_t, sem).wait()
            pltpu.sync_copy(rows_t, out_hbm.at[pl.ds(off, tail)])

    return pl.kernel(
        body,
        mesh=mesh,
        out_type=jax.ShapeDtypeStruct((n_idx, d), _F32),
        scratch_types=scratch,
        compiler_params=pltpu.CompilerParams(use_tc_tiling_on_sc=False),
    )


def _p1_body(raw_ref, wrel_ref, brel_ref, s_ref, ss_ref):
    y = jnp.dot(raw_ref[...], wrel_ref[...], preferred_element_type=_F32)
    y = y + brel_ref[...]

    @pl.when(pl.program_id(0) == 0)
    def _init():
        s_ref[...] = jnp.zeros(s_ref.shape, s_ref.dtype)
        ss_ref[...] = jnp.zeros(ss_ref.shape, ss_ref.dtype)

    s_ref[...] += jnp.sum(y, axis=0, keepdims=True)
    ss_ref[...] += jnp.sum(y * y, axis=0, keepdims=True)


def _rel_stats(raw2, W_rel, b_rel2, rb):
    nk, ori = raw2.shape
    crel = W_rel.shape[1]
    steps = nk // rb
    assert steps * rb == nk
    return pl.pallas_call(
        _p1_body,
        grid=(steps,),
        in_specs=[
            pl.BlockSpec((rb, ori), lambda i: (i, 0)),
            pl.BlockSpec((ori, crel), lambda i: (0, 0)),
            pl.BlockSpec((1, crel), lambda i: (0, 0)),
        ],
        out_specs=[pl.BlockSpec((1, crel), lambda i: (0, 0))] * 2,
        out_shape=[jax.ShapeDtypeStruct((1, crel), _F32)] * 2,
    )(raw2, W_rel, b_rel2)


def _p2_body(g_ref, raw_ref, f_ref, srel_ref, ssrel_ref,
             wrel_ref, brel_ref, grel_ref, berel_ref,
             wgg_ref, wrg_ref, wgr_ref, wrr_ref,
             wog_ref, wor_ref, bo_ref, wsc_ref, bsc_ref,
             out_ref, sc_ref, so_ref, sso_ref, ssc_ref, sssc_ref,
             *, blk, k, m_rel):
    inv_m = 1.0 / m_rel
    mean = srel_ref[...] * inv_m
    var = ssrel_ref[...] * inv_m - mean * mean
    scale = lax.rsqrt(var + _EPS) * grel_ref[...]
    shift = berel_ref[...] - mean * scale

    rl = jnp.dot(raw_ref[...], wrel_ref[...], preferred_element_type=_F32)
    rl = rl + brel_ref[...]
    rel = _leaky(rl * scale + shift)
    g = g_ref[...]

    lg = (jnp.dot(g, wgg_ref[...], preferred_element_type=_F32)
          + jnp.dot(rel, wrg_ref[...], preferred_element_type=_F32))
    lr = (jnp.dot(g, wgr_ref[...], preferred_element_type=_F32)
          + jnp.dot(rel, wrr_ref[...], preferred_element_type=_F32))
    eg = jnp.exp(lg)
    er = jnp.exp(lr)

    cg = g.shape[1]
    cr = rel.shape[1]
    g3 = g.reshape(blk, k, cg)
    r3 = rel.reshape(blk, k, cr)
    eg3 = eg.reshape(blk, k, cg)
    er3 = er.reshape(blk, k, cr)
    pg = jnp.sum(eg3 * g3, axis=1) / jnp.sum(eg3, axis=1)
    pr = jnp.sum(er3 * r3, axis=1) / jnp.sum(er3, axis=1)

    out_lin = (jnp.dot(pg, wog_ref[...], preferred_element_type=_F32)
               + jnp.dot(pr, wor_ref[...], preferred_element_type=_F32)
               + bo_ref[...])
    sc_lin = jnp.dot(f_ref[...], wsc_ref[...], preferred_element_type=_F32)
    sc_lin = sc_lin + bsc_ref[...]

    out_ref[...] = out_lin
    sc_ref[...] = sc_lin

    @pl.when(pl.program_id(0) == 0)
    def _init():
        for r in (so_ref, sso_ref, ssc_ref, sssc_ref):
            r[...] = jnp.zeros(r.shape, r.dtype)

    so_ref[...] += jnp.sum(out_lin, axis=0, keepdims=True)
    sso_ref[...] += jnp.sum(out_lin * out_lin, axis=0, keepdims=True)
    ssc_ref[...] += jnp.sum(sc_lin, axis=0, keepdims=True)
    sssc_ref[...] += jnp.sum(sc_lin * sc_lin, axis=0, keepdims=True)


def _main_pass(gathered, raw2, feat2, s_rel, ss_rel,
               W_rel, b_rel2, g_rel2, be_rel2,
               wgg, wrg, wgr, wrr, wog, wor, b_out2, W_sc, b_sc2, blk):
    n, cin = feat2.shape
    nk, ori = raw2.shape
    k = nk // n
    crel = W_rel.shape[1]
    cout = wog.shape[1]
    steps = n // blk
    assert steps * blk == n
    blkk = blk * k
    body = functools.partial(_p2_body, blk=blk, k=k, m_rel=float(nk))
    full = lambda a: pl.BlockSpec(a.shape, lambda i: (0,) * a.ndim)
    return pl.pallas_call(
        body,
        grid=(steps,),
        in_specs=[
            pl.BlockSpec((blkk, cin), lambda i: (i, 0)),
            pl.BlockSpec((blkk, ori), lambda i: (i, 0)),
            pl.BlockSpec((blk, cin), lambda i: (i, 0)),
            full(s_rel), full(ss_rel),
            full(W_rel), full(b_rel2), full(g_rel2), full(be_rel2),
            full(wgg), full(wrg), full(wgr), full(wrr),
            full(wog), full(wor), full(b_out2), full(W_sc), full(b_sc2),
        ],
        out_specs=[
            pl.BlockSpec((blk, cout), lambda i: (i, 0)),
            pl.BlockSpec((blk, cout), lambda i: (i, 0)),
            pl.BlockSpec((1, cout), lambda i: (0, 0)),
            pl.BlockSpec((1, cout), lambda i: (0, 0)),
            pl.BlockSpec((1, cout), lambda i: (0, 0)),
            pl.BlockSpec((1, cout), lambda i: (0, 0)),
        ],
        out_shape=[
            jax.ShapeDtypeStruct((n, cout), _F32),
            jax.ShapeDtypeStruct((n, cout), _F32),
            jax.ShapeDtypeStruct((1, cout), _F32),
            jax.ShapeDtypeStruct((1, cout), _F32),
            jax.ShapeDtypeStruct((1, cout), _F32),
            jax.ShapeDtypeStruct((1, cout), _F32),
        ],
    )(gathered, raw2, feat2, s_rel, ss_rel,
      W_rel, b_rel2, g_rel2, be_rel2,
      wgg, wrg, wgr, wrr, wog, wor, b_out2, W_sc, b_sc2)


def _p3_body(o_ref, s_ref, so_ref, sso_ref, ssc_ref, sssc_ref,
             go_ref, beo_ref, gs_ref, bes_ref, y_ref, *, n):
    inv_n = 1.0 / n
    mo = so_ref[...] * inv_n
    vo = sso_ref[...] * inv_n - mo * mo
    ao = lax.rsqrt(vo + _EPS) * go_ref[...]
    ms = ssc_ref[...] * inv_n
    vs = sssc_ref[...] * inv_n - ms * ms
    as_ = lax.rsqrt(vs + _EPS) * gs_ref[...]
    y = ((o_ref[...] - mo) * ao + beo_ref[...]
         + (s_ref[...] - ms) * as_ + bes_ref[...])
    y_ref[...] = _leaky(y)


def _finalize(out_lin, sc_lin, so, sso, ssc, sssc,
              g_out2, be_out2, g_sc2, be_sc2, blk):
    n, cout = out_lin.shape
    steps = n // blk
    assert steps * blk == n
    body = functools.partial(_p3_body, n=float(n))
    full = lambda a: pl.BlockSpec(a.shape, lambda i: (0,) * a.ndim)
    return pl.pallas_call(
        body,
        grid=(steps,),
        in_specs=[
            pl.BlockSpec((blk, cout), lambda i: (i, 0)),
            pl.BlockSpec((blk, cout), lambda i: (i, 0)),
            full(so), full(sso), full(ssc), full(sssc),
            full(g_out2), full(be_out2), full(g_sc2), full(be_sc2),
        ],
        out_specs=pl.BlockSpec((blk, cout), lambda i: (i, 0)),
        out_shape=jax.ShapeDtypeStruct((n, cout), _F32),
    )(out_lin, sc_lin, so, sso, ssc, sssc,
      g_out2, be_out2, g_sc2, be_sc2)


def kernel(xyz, feature, raw_relative_feature, neighbors_idx,
           W_rel, b_rel, g_rel, be_rel, W_attn,
           W_out, b_out, g_out, be_out, W_sc, b_sc, g_sc, be_sc):
    bsz, n, cin = feature.shape
    k = neighbors_idx.shape[-1]
    ori = raw_relative_feature.shape[-1]
    cout = W_out.shape[1]

    feat2 = feature.reshape(n, cin)
    idx_flat = neighbors_idx.reshape(n * k)
    raw2 = raw_relative_feature.reshape(n * k, ori)

    gathered = _make_sc_gather(n * k, cin)(feat2, idx_flat)

    b_rel2 = b_rel.reshape(1, -1)
    g_rel2 = g_rel.reshape(1, -1)
    be_rel2 = be_rel.reshape(1, -1)
    b_out2 = b_out.reshape(1, -1)
    g_out2 = g_out.reshape(1, -1)
    be_out2 = be_out.reshape(1, -1)
    b_sc2 = b_sc.reshape(1, -1)
    g_sc2 = g_sc.reshape(1, -1)
    be_sc2 = be_sc.reshape(1, -1)
    wgg = W_attn[:cin, :cin]
    wrg = W_attn[cin:, :cin]
    wgr = W_attn[:cin, cin:]
    wrr = W_attn[cin:, cin:]
    wog = W_out[:cin]
    wor = W_out[cin:]

    s_rel, ss_rel = _rel_stats(raw2, W_rel, b_rel2, rb=32000)
    out_lin, sc_lin, so, sso, ssc, sssc = _main_pass(
        gathered, raw2, feat2, s_rel, ss_rel,
        W_rel, b_rel2, g_rel2, be_rel2,
        wgg, wrg, wgr, wrr, wog, wor, b_out2, W_sc, b_sc2, blk=400)
    y = _finalize(out_lin, sc_lin, so, sso, ssc, sssc,
                  g_out2, be_out2, g_sc2, be_sc2, blk=5000)
    return (xyz, y.reshape(bsz, n, cout), raw_relative_feature, neighbors_idx)
